# baseline (speedup 1.0000x reference)
import functools

import jax
import jax.numpy as jnp
from jax import lax
from jax.experimental import pallas as pl
from jax.experimental.pallas import tpu as pltpu
from jax.experimental.pallas import tpu_sc as plsc

N_NODES = 10000
T = 8
E = 640000
TT = T * T

NC = 2
NS = 16
NW = NC * NS
EPW = E // NW
C = 400
NCHUNK = EPW // C
G = C // 16

_mesh = plsc.VectorSubcoreMesh(
    core_axis_name="c", subcore_axis_name="s", num_cores=NC, num_subcores=NS
)


@functools.partial(
    pl.kernel,
    out_type=jax.ShapeDtypeStruct((E * TT,), jnp.float32),
    mesh=_mesh,
    compiler_params=pltpu.CompilerParams(needs_layout_passes=False),
    scratch_types=[
        pltpu.VMEM((N_NODES * T,), jnp.float32),
        pltpu.VMEM((C,), jnp.int32),
        pltpu.VMEM((C,), jnp.int32),
        pltpu.VMEM((C * TT,), jnp.float32),
    ],
)
def _encode(src_hbm, dst_hbm, node_hbm, out_hbm, table_v, idx1_v, idx2_v, out_v):
    wid = lax.axis_index("s") * NC + lax.axis_index("c")
    pltpu.sync_copy(node_hbm, table_v)

    def chunk_body(k, _):
        base = wid * EPW + k * C
        pltpu.sync_copy(src_hbm.at[pl.ds(base, C)], idx1_v)
        pltpu.sync_copy(dst_hbm.at[pl.ds(base, C)], idx2_v)

        @plsc.parallel_loop(0, G, 1, unroll=2)
        def group_body(g):
            lane = lax.iota(jnp.int32, 16)
            pat_b = lane % T
            pat_a = [2 * r + lane // T for r in range(4)]
            vs8 = idx1_v[pl.ds(g * 16, 16)] * T
            vd8 = idx2_v[pl.ds(g * 16, 16)] * T
            obase = g * (16 * TT)
            for l in range(16):
                sb = jnp.full((16,), vs8[l], jnp.int32)
                db = jnp.full((16,), vd8[l], jnp.int32)
                b = plsc.load_gather(table_v, [db + pat_b])
                for r in range(4):
                    a = plsc.load_gather(table_v, [sb + pat_a[r]])
                    out_v[pl.ds(obase + l * TT + r * 16, 16)] = a * b

        pltpu.sync_copy(out_v, out_hbm.at[pl.ds(base * TT, C * TT)])
        return 0

    lax.fori_loop(0, NCHUNK, chunk_body, 0)


def kernel(edge_index, node_type):
    src = edge_index[0]
    dst = edge_index[1]
    out = _encode(src, dst, node_type.reshape(-1))
    return out.reshape(E, TT)


# speedup vs baseline: 1.5634x; 1.5634x over previous
import functools

import jax
import jax.numpy as jnp
from jax import lax
from jax.experimental import pallas as pl
from jax.experimental.pallas import tpu as pltpu
from jax.experimental.pallas import tpu_sc as plsc

N_NODES = 10000
T = 8
E = 640000
TT = T * T

NC = 2
NS = 16
NW = NC * NS
EPW = E // NW
C = 400
NCHUNK = EPW // C
G = C // 16

_mesh = plsc.VectorSubcoreMesh(
    core_axis_name="c", subcore_axis_name="s", num_cores=NC, num_subcores=NS
)


@functools.partial(
    pl.kernel,
    out_type=jax.ShapeDtypeStruct((E, TT), jnp.float32),
    mesh=_mesh,
    compiler_params=pltpu.CompilerParams(needs_layout_passes=False, use_tc_tiling_on_sc=False),
    scratch_types=[
        pltpu.VMEM((N_NODES * T,), jnp.float32),
        pltpu.VMEM((C,), jnp.int32),
        pltpu.VMEM((C,), jnp.int32),
        pltpu.VMEM((C, TT), jnp.float32),
    ],
)
def _encode(src_hbm, dst_hbm, node_hbm, out_hbm, table_v, idx1_v, idx2_v, out_v):
    wid = lax.axis_index("s") * NC + lax.axis_index("c")
    pltpu.sync_copy(node_hbm, table_v)

    def chunk_body(k, _):
        base = wid * EPW + k * C
        pltpu.sync_copy(src_hbm.at[pl.ds(base, C)], idx1_v)
        pltpu.sync_copy(dst_hbm.at[pl.ds(base, C)], idx2_v)

        @plsc.parallel_loop(0, G, 1, unroll=1)
        def group_body(g):
            lane = lax.iota(jnp.int32, 16)
            pat_b = lane % T
            pat_a = [2 * r + lane // T for r in range(4)]
            vs8 = idx1_v[pl.ds(g * 16, 16)] * T
            vd8 = idx2_v[pl.ds(g * 16, 16)] * T
            obase = g * 16
            for l in range(16):
                sb = jnp.full((16,), vs8[l], jnp.int32)
                db = jnp.full((16,), vd8[l], jnp.int32)
                b = plsc.load_gather(table_v, [db + pat_b])
                for r in range(4):
                    a = plsc.load_gather(table_v, [sb + pat_a[r]])
                    out_v[obase + l, pl.ds(r * 16, 16)] = a * b

        pltpu.sync_copy(out_v, out_hbm.at[pl.ds(base, C)])
        return 0

    lax.fori_loop(0, NCHUNK, chunk_body, 0)


def kernel(edge_index, node_type):
    src = edge_index[0]
    dst = edge_index[1]
    return _encode(src, dst, node_type.reshape(-1))
